# baseline (speedup 1.0000x reference)
import functools

import jax
import jax.numpy as jnp
from jax import lax
from jax.experimental import pallas as pl
from jax.experimental.pallas import tpu as pltpu
from jax.experimental.pallas import tpu_sc as plsc


_RC = 16
_VEC = 16


def _make_sc_kernel(bs, seq_len, ch, dtype):
    info = plsc.get_sparse_core_info()
    nc, ns = info.num_cores, info.num_subcores
    nw = nc * ns
    seq_per_w = seq_len // nw
    n_chunks = seq_per_w // _RC
    n_steps = n_chunks * bs
    vecs_per_row = ch // _VEC
    mesh = plsc.VectorSubcoreMesh(core_axis_name="c", subcore_axis_name="s")

    @functools.partial(
        pl.kernel,
        mesh=mesh,
        out_type=jax.ShapeDtypeStruct((bs * seq_len, ch), dtype),
        scratch_types=[
            pltpu.VMEM((_RC, ch), dtype),
            pltpu.VMEM((_RC, ch), dtype),
            pltpu.VMEM((_RC, ch), dtype),
            pltpu.VMEM((_RC, ch), dtype),
            pltpu.SemaphoreType.DMA,
            pltpu.SemaphoreType.DMA,
            pltpu.SemaphoreType.DMA,
            pltpu.SemaphoreType.DMA,
            pltpu.SemaphoreType.DMA,
            pltpu.SemaphoreType.DMA,
        ],
    )
    def sc_kernel(x_hbm, emb_hbm, out_hbm, xb0, xb1, eb0, eb1,
                  xs0, xs1, es0, es1, os0, os1):
        xbufs, ebufs = (xb0, xb1), (eb0, eb1)
        xsems, esems, osems = (xs0, xs1), (es0, es1), (os0, os1)
        wid = lax.axis_index("s") * nc + lax.axis_index("c")
        s0 = wid * seq_per_w

        def x_row(step):
            c, b = step // bs, step % bs
            return b * seq_len + s0 + c * _RC

        def add_chunk(xb, eb):
            def body(it, _):
                r = it // (vecs_per_row // 4)
                j0 = (it % (vecs_per_row // 4)) * 4
                for u in range(4):
                    col = (j0 + u) * _VEC
                    xb[r, pl.ds(col, _VEC)] = (
                        xb[r, pl.ds(col, _VEC)] + eb[r, pl.ds(col, _VEC)])
                return _
            lax.fori_loop(0, _RC * vecs_per_row // 4, body, None)

        lds = [None] * n_steps
        sts = [None] * n_steps
        elds = [None] * n_chunks
        elds[0] = pltpu.async_copy(
            emb_hbm.at[pl.ds(s0, _RC)], ebufs[0], esems[0])
        lds[0] = pltpu.async_copy(
            x_hbm.at[pl.ds(x_row(0), _RC)], xbufs[0], xsems[0])
        for s in range(n_steps):
            cur = s & 1
            c = s // bs
            if s % bs == 0 and c + 1 < n_chunks:
                elds[c + 1] = pltpu.async_copy(
                    emb_hbm.at[pl.ds(s0 + (c + 1) * _RC, _RC)],
                    ebufs[(c + 1) & 1], esems[(c + 1) & 1])
            if s + 1 < n_steps:
                nxt = (s + 1) & 1
                if s >= 1:
                    sts[s - 1].wait()
                lds[s + 1] = pltpu.async_copy(
                    x_hbm.at[pl.ds(x_row(s + 1), _RC)], xbufs[nxt],
                    xsems[nxt])
            if s % bs == 0:
                elds[c].wait()
            lds[s].wait()
            add_chunk(xbufs[cur], ebufs[c & 1])
            sts[s] = pltpu.async_copy(
                xbufs[cur], out_hbm.at[pl.ds(x_row(s), _RC)], osems[cur])
        sts[n_steps - 2].wait()
        sts[n_steps - 1].wait()

    return sc_kernel


def kernel(x, emb_weight):
    bs, seq_len, ch = x.shape
    emb = emb_weight[:seq_len]
    x2 = x.reshape(bs * seq_len, ch)
    sc = _make_sc_kernel(bs, seq_len, ch, x.dtype)
    return sc(x2, emb).reshape(bs, seq_len, ch)


# speedup vs baseline: 1.6637x; 1.6637x over previous
import jax
import jax.numpy as jnp
from jax.experimental import pallas as pl


_SEQ_BLOCK = 256


def _add_kernel(x_ref, emb_ref, out_ref):
    out_ref[...] = x_ref[...] + emb_ref[...][None, :, :]


def kernel(x, emb_weight):
    bs, seq_len, ch = x.shape
    emb = emb_weight[:seq_len]
    blk = _SEQ_BLOCK if seq_len % _SEQ_BLOCK == 0 else seq_len
    grid = (seq_len // blk,)
    return pl.pallas_call(
        _add_kernel,
        grid=grid,
        in_specs=[
            pl.BlockSpec((bs, blk, ch), lambda i: (0, i, 0)),
            pl.BlockSpec((blk, ch), lambda i: (i, 0)),
        ],
        out_specs=pl.BlockSpec((bs, blk, ch), lambda i: (0, i, 0)),
        out_shape=jax.ShapeDtypeStruct((bs, seq_len, ch), x.dtype),
    )(x, emb)


# speedup vs baseline: 1.6669x; 1.0019x over previous
import jax
import jax.numpy as jnp
from jax.experimental import pallas as pl


_SEQ_BLOCK = 512


def _add_kernel(x_ref, emb_ref, out_ref):
    out_ref[...] = x_ref[...] + emb_ref[...][None, :, :]


def kernel(x, emb_weight):
    bs, seq_len, ch = x.shape
    emb = emb_weight[:seq_len]
    blk = _SEQ_BLOCK if seq_len % _SEQ_BLOCK == 0 else seq_len
    grid = (seq_len // blk,)
    return pl.pallas_call(
        _add_kernel,
        grid=grid,
        in_specs=[
            pl.BlockSpec((bs, blk, ch), lambda i: (0, i, 0)),
            pl.BlockSpec((blk, ch), lambda i: (i, 0)),
        ],
        out_specs=pl.BlockSpec((bs, blk, ch), lambda i: (0, i, 0)),
        out_shape=jax.ShapeDtypeStruct((bs, seq_len, ch), x.dtype),
    )(x, emb)
